# baseline (device time: 90114 ns/iter reference)
import jax
import jax.numpy as jnp
from jax import lax
from jax.experimental import pallas as pl
from jax.experimental.pallas import tpu as pltpu

N_DEV = 16
NQ = 4


def kernel(x, W1, W2):
    m, k = x.shape
    _, h = W1.shape
    _, n = W2.shape
    chunk = m // NQ
    half = chunk // 2

    def body(x_ref, w1_ref, w2_ref, out_ref,
             ptop, pbot, ctop, cbot, zb1, zb2,
             sp_s, sp_r, sm_s, sm_r,
             za_s, za_r, zb_s, zb_r,
             ga_s, ga_r, gb_s, gb_r,
             ap_s, ap_r, am_s, am_r):
        me = lax.axis_index("i")
        z = me // NQ
        q = me % NQ
        right = z * NQ + (q + 1) % NQ
        left = z * NQ + (q - 1) % NQ
        w1p = me ^ 4
        w2p = me ^ 8

        barrier_sem = pltpu.get_barrier_semaphore()
        for nbr in [left, right, w1p, w2p]:
            pl.semaphore_signal(
                barrier_sem, inc=1,
                device_id=(nbr,), device_id_type=pl.DeviceIdType.MESH,
            )
        pl.semaphore_wait(barrier_sem, 4)

        def compute_top(c):
            xs = x_ref[pl.ds(c * chunk, half), :]
            hm = jnp.maximum(
                jnp.dot(xs, w1_ref[...], preferred_element_type=jnp.float32), 0.0)
            ptop[c] = jnp.dot(hm, w2_ref[...], preferred_element_type=jnp.float32)

        def compute_bot(c):
            xs = x_ref[pl.ds(c * chunk + half, half), :]
            hm = jnp.maximum(
                jnp.dot(xs, w1_ref[...], preferred_element_type=jnp.float32), 0.0)
            pbot[c] = jnp.dot(hm, w2_ref[...], preferred_element_type=jnp.float32)

        compute_top(q)

        for s in range(NQ - 1):
            st = (q - s) % NQ
            rt = (q - s - 1) % NQ
            sb = (q + s) % NQ
            rb = (q + s + 1) % NQ
            rp = pltpu.make_async_remote_copy(
                src_ref=ptop.at[st], dst_ref=ctop.at[s],
                send_sem=sp_s.at[s], recv_sem=sp_r.at[s],
                device_id=(right,), device_id_type=pl.DeviceIdType.MESH,
            )
            rm = pltpu.make_async_remote_copy(
                src_ref=pbot.at[sb], dst_ref=cbot.at[s],
                send_sem=sm_s.at[s], recv_sem=sm_r.at[s],
                device_id=(left,), device_id_type=pl.DeviceIdType.MESH,
            )
            rp.start()
            if s == 0:
                compute_bot(q)
            rm.start()
            if s == 0:
                compute_top((q - 1) % NQ)
                compute_bot((q + 1) % NQ)
            if s < NQ - 2:
                compute_top((q - s - 2) % NQ)
                compute_bot((q + s + 2) % NQ)
            rp.wait_recv()
            rm.wait_recv()
            ptop[rt] = ptop[rt] + ctop[s]
            pbot[rb] = pbot[rb] + cbot[s]
            rp.wait_send()
            rm.wait_send()

        ownt = (q + 1) % NQ
        ownb = (q - 1) % NQ

        koff = jnp.where(z % 2 == 0, 0, half // 2)
        soff = (half // 2) - koff
        e1t = pltpu.make_async_remote_copy(
            src_ref=ptop.at[ownt, pl.ds(soff, half // 2), :],
            dst_ref=zb1.at[0],
            send_sem=za_s.at[0], recv_sem=za_r.at[0],
            device_id=(w1p,), device_id_type=pl.DeviceIdType.MESH,
        )
        e1b = pltpu.make_async_remote_copy(
            src_ref=pbot.at[ownb, pl.ds(soff, half // 2), :],
            dst_ref=zb1.at[1],
            send_sem=za_s.at[1], recv_sem=za_r.at[1],
            device_id=(w1p,), device_id_type=pl.DeviceIdType.MESH,
        )
        e1t.start()
        e1b.start()
        e1t.wait_recv()
        e1b.wait_recv()
        ptop[ownt, pl.ds(koff, half // 2), :] = (
            ptop[ownt, pl.ds(koff, half // 2), :] + zb1[0])
        pbot[ownb, pl.ds(koff, half // 2), :] = (
            pbot[ownb, pl.ds(koff, half // 2), :] + zb1[1])

        k2 = jnp.where((z // 2) % 2 == 0, 0, half // 4)
        s2 = (half // 4) - k2
        e2t = pltpu.make_async_remote_copy(
            src_ref=ptop.at[ownt, pl.ds(koff + s2, half // 4), :],
            dst_ref=zb2.at[0],
            send_sem=zb_s.at[0], recv_sem=zb_r.at[0],
            device_id=(w2p,), device_id_type=pl.DeviceIdType.MESH,
        )
        e2b = pltpu.make_async_remote_copy(
            src_ref=pbot.at[ownb, pl.ds(koff + s2, half // 4), :],
            dst_ref=zb2.at[1],
            send_sem=zb_s.at[1], recv_sem=zb_r.at[1],
            device_id=(w2p,), device_id_type=pl.DeviceIdType.MESH,
        )
        e2t.start()
        e2b.start()
        e2t.wait_recv()
        e2b.wait_recv()
        ptop[ownt, pl.ds(koff + k2, half // 4), :] = (
            ptop[ownt, pl.ds(koff + k2, half // 4), :] + zb2[0])
        pbot[ownb, pl.ds(koff + k2, half // 4), :] = (
            pbot[ownb, pl.ds(koff + k2, half // 4), :] + zb2[1])

        trow = ownt * chunk + koff + k2
        brow = ownb * chunk + half + koff + k2
        out_ref[pl.ds(trow, half // 4), :] = ptop[ownt, pl.ds(koff + k2, half // 4), :]
        out_ref[pl.ds(brow, half // 4), :] = pbot[ownb, pl.ds(koff + k2, half // 4), :]

        g2t = pltpu.make_async_remote_copy(
            src_ref=out_ref.at[pl.ds(trow, half // 4), :],
            dst_ref=out_ref.at[pl.ds(trow, half // 4), :],
            send_sem=ga_s.at[0], recv_sem=ga_r.at[0],
            device_id=(w2p,), device_id_type=pl.DeviceIdType.MESH,
        )
        g2b = pltpu.make_async_remote_copy(
            src_ref=out_ref.at[pl.ds(brow, half // 4), :],
            dst_ref=out_ref.at[pl.ds(brow, half // 4), :],
            send_sem=ga_s.at[1], recv_sem=ga_r.at[1],
            device_id=(w2p,), device_id_type=pl.DeviceIdType.MESH,
        )
        g2t.start()
        g2b.start()
        g2t.wait_recv()
        g2b.wait_recv()

        t64 = ownt * chunk + koff
        b64 = ownb * chunk + half + koff
        g1t = pltpu.make_async_remote_copy(
            src_ref=out_ref.at[pl.ds(t64, half // 2), :],
            dst_ref=out_ref.at[pl.ds(t64, half // 2), :],
            send_sem=gb_s.at[0], recv_sem=gb_r.at[0],
            device_id=(w1p,), device_id_type=pl.DeviceIdType.MESH,
        )
        g1b = pltpu.make_async_remote_copy(
            src_ref=out_ref.at[pl.ds(b64, half // 2), :],
            dst_ref=out_ref.at[pl.ds(b64, half // 2), :],
            send_sem=gb_s.at[1], recv_sem=gb_r.at[1],
            device_id=(w1p,), device_id_type=pl.DeviceIdType.MESH,
        )
        g1t.start()
        g1b.start()
        g1t.wait_recv()
        g1b.wait_recv()

        for t in range(NQ - 1):
            ct_ = (q + 1 - t) % NQ
            cb_ = (q - 1 + t) % NQ
            rp = pltpu.make_async_remote_copy(
                src_ref=out_ref.at[pl.ds(ct_ * chunk, half), :],
                dst_ref=out_ref.at[pl.ds(ct_ * chunk, half), :],
                send_sem=ap_s.at[t], recv_sem=ap_r.at[t],
                device_id=(right,), device_id_type=pl.DeviceIdType.MESH,
            )
            rm = pltpu.make_async_remote_copy(
                src_ref=out_ref.at[pl.ds(cb_ * chunk + half, half), :],
                dst_ref=out_ref.at[pl.ds(cb_ * chunk + half, half), :],
                send_sem=am_s.at[t], recv_sem=am_r.at[t],
                device_id=(left,), device_id_type=pl.DeviceIdType.MESH,
            )
            rp.start()
            rm.start()
            rp.wait_recv()
            rm.wait_recv()
            rp.wait_send()
            rm.wait_send()

        for d in [e1t, e1b, e2t, e2b, g2t, g2b, g1t, g1b]:
            d.wait_send()

    nsteps = NQ - 1
    return pl.pallas_call(
        body,
        out_shape=jax.ShapeDtypeStruct((m, n), jnp.float32),
        in_specs=[
            pl.BlockSpec(memory_space=pltpu.VMEM),
            pl.BlockSpec(memory_space=pltpu.VMEM),
            pl.BlockSpec(memory_space=pltpu.VMEM),
        ],
        out_specs=pl.BlockSpec(memory_space=pltpu.VMEM),
        scratch_shapes=[
            pltpu.VMEM((NQ, half, n), jnp.float32),
            pltpu.VMEM((NQ, half, n), jnp.float32),
            pltpu.VMEM((nsteps, half, n), jnp.float32),
            pltpu.VMEM((nsteps, half, n), jnp.float32),
            pltpu.VMEM((2, half // 2, n), jnp.float32),
            pltpu.VMEM((2, half // 4, n), jnp.float32),
            pltpu.SemaphoreType.DMA((nsteps,)),
            pltpu.SemaphoreType.DMA((nsteps,)),
            pltpu.SemaphoreType.DMA((nsteps,)),
            pltpu.SemaphoreType.DMA((nsteps,)),
            pltpu.SemaphoreType.DMA((2,)),
            pltpu.SemaphoreType.DMA((2,)),
            pltpu.SemaphoreType.DMA((2,)),
            pltpu.SemaphoreType.DMA((2,)),
            pltpu.SemaphoreType.DMA((2,)),
            pltpu.SemaphoreType.DMA((2,)),
            pltpu.SemaphoreType.DMA((2,)),
            pltpu.SemaphoreType.DMA((2,)),
            pltpu.SemaphoreType.DMA((nsteps,)),
            pltpu.SemaphoreType.DMA((nsteps,)),
            pltpu.SemaphoreType.DMA((nsteps,)),
            pltpu.SemaphoreType.DMA((nsteps,)),
        ],
        compiler_params=pltpu.CompilerParams(collective_id=0),
    )(x, W1, W2)


# device time: 88752 ns/iter; 1.0153x vs baseline; 1.0153x over previous
import jax
import jax.numpy as jnp
from jax import lax
from jax.experimental import pallas as pl
from jax.experimental.pallas import tpu as pltpu

N_DEV = 16
NQ = 4


def kernel(x, W1, W2):
    m, k = x.shape
    _, h = W1.shape
    _, n = W2.shape
    chunk = m // NQ
    half = chunk // 2

    def body(x_ref, w1_ref, w2_ref, out_ref,
             ptop, pbot, ctop, cbot, zb1, zb2,
             sp_s, sp_r, sm_s, sm_r,
             za_s, za_r, zb_s, zb_r,
             ga_s, ga_r, gb_s, gb_r,
             ap_s, ap_r, am_s, am_r):
        me = lax.axis_index("i")
        z = me // NQ
        q = me % NQ
        right = z * NQ + (q + 1) % NQ
        left = z * NQ + (q - 1) % NQ
        w1p = me ^ 4
        w2p = me ^ 8

        barrier_sem = pltpu.get_barrier_semaphore()
        for nbr in [left, right, w1p, w2p]:
            pl.semaphore_signal(
                barrier_sem, inc=1,
                device_id=(nbr,), device_id_type=pl.DeviceIdType.MESH,
            )
        pl.semaphore_wait(barrier_sem, 4)

        def compute_top(c):
            xs = x_ref[pl.ds(c * chunk, half), :]
            hm = jnp.maximum(
                jnp.dot(xs, w1_ref[...], preferred_element_type=jnp.float32), 0.0)
            ptop[c] = jnp.dot(hm, w2_ref[...], preferred_element_type=jnp.float32)

        def compute_bot(c):
            xs = x_ref[pl.ds(c * chunk + half, half), :]
            hm = jnp.maximum(
                jnp.dot(xs, w1_ref[...], preferred_element_type=jnp.float32), 0.0)
            pbot[c] = jnp.dot(hm, w2_ref[...], preferred_element_type=jnp.float32)

        compute_top(q)

        for s in range(NQ - 1):
            st = (q - s) % NQ
            rt = (q - s - 1) % NQ
            sb = (q + s) % NQ
            rb = (q + s + 1) % NQ
            rp = pltpu.make_async_remote_copy(
                src_ref=ptop.at[st], dst_ref=ctop.at[s],
                send_sem=sp_s.at[s], recv_sem=sp_r.at[s],
                device_id=(right,), device_id_type=pl.DeviceIdType.MESH,
            )
            rm = pltpu.make_async_remote_copy(
                src_ref=pbot.at[sb], dst_ref=cbot.at[s],
                send_sem=sm_s.at[s], recv_sem=sm_r.at[s],
                device_id=(left,), device_id_type=pl.DeviceIdType.MESH,
            )
            rp.start()
            if s == 0:
                compute_bot(q)
            rm.start()
            if s == 0:
                compute_top((q - 1) % NQ)
                compute_bot((q + 1) % NQ)
            if s < NQ - 2:
                compute_top((q - s - 2) % NQ)
                compute_bot((q + s + 2) % NQ)
            rp.wait_recv()
            rm.wait_recv()
            ptop[rt] = ptop[rt] + ctop[s]
            pbot[rb] = pbot[rb] + cbot[s]
            rp.wait_send()
            rm.wait_send()

        ownt = (q + 1) % NQ
        ownb = (q - 1) % NQ

        koff = jnp.where(z % 2 == 0, 0, half // 2)
        soff = (half // 2) - koff
        e1t = pltpu.make_async_remote_copy(
            src_ref=ptop.at[ownt, pl.ds(soff, half // 2), :],
            dst_ref=zb1.at[0],
            send_sem=za_s.at[0], recv_sem=za_r.at[0],
            device_id=(w1p,), device_id_type=pl.DeviceIdType.MESH,
        )
        e1b = pltpu.make_async_remote_copy(
            src_ref=pbot.at[ownb, pl.ds(soff, half // 2), :],
            dst_ref=zb1.at[1],
            send_sem=za_s.at[1], recv_sem=za_r.at[1],
            device_id=(w1p,), device_id_type=pl.DeviceIdType.MESH,
        )
        e1t.start()
        e1b.start()
        e1t.wait_recv()
        e1b.wait_recv()
        ptop[ownt, pl.ds(koff, half // 2), :] = (
            ptop[ownt, pl.ds(koff, half // 2), :] + zb1[0])
        pbot[ownb, pl.ds(koff, half // 2), :] = (
            pbot[ownb, pl.ds(koff, half // 2), :] + zb1[1])

        e2t = pltpu.make_async_remote_copy(
            src_ref=ptop.at[ownt, pl.ds(koff, half // 2), :],
            dst_ref=zb2.at[0],
            send_sem=zb_s.at[0], recv_sem=zb_r.at[0],
            device_id=(w2p,), device_id_type=pl.DeviceIdType.MESH,
        )
        e2b = pltpu.make_async_remote_copy(
            src_ref=pbot.at[ownb, pl.ds(koff, half // 2), :],
            dst_ref=zb2.at[1],
            send_sem=zb_s.at[1], recv_sem=zb_r.at[1],
            device_id=(w2p,), device_id_type=pl.DeviceIdType.MESH,
        )
        e2t.start()
        e2b.start()
        e2t.wait_recv()
        e2b.wait_recv()
        ptop[ownt, pl.ds(koff, half // 2), :] = (
            ptop[ownt, pl.ds(koff, half // 2), :] + zb2[0])
        pbot[ownb, pl.ds(koff, half // 2), :] = (
            pbot[ownb, pl.ds(koff, half // 2), :] + zb2[1])

        trow = ownt * chunk + koff
        brow = ownb * chunk + half + koff
        out_ref[pl.ds(trow, half // 2), :] = ptop[ownt, pl.ds(koff, half // 2), :]
        out_ref[pl.ds(brow, half // 2), :] = pbot[ownb, pl.ds(koff, half // 2), :]

        t64 = ownt * chunk + koff
        b64 = ownb * chunk + half + koff
        g1t = pltpu.make_async_remote_copy(
            src_ref=out_ref.at[pl.ds(t64, half // 2), :],
            dst_ref=out_ref.at[pl.ds(t64, half // 2), :],
            send_sem=gb_s.at[0], recv_sem=gb_r.at[0],
            device_id=(w1p,), device_id_type=pl.DeviceIdType.MESH,
        )
        g1b = pltpu.make_async_remote_copy(
            src_ref=out_ref.at[pl.ds(b64, half // 2), :],
            dst_ref=out_ref.at[pl.ds(b64, half // 2), :],
            send_sem=gb_s.at[1], recv_sem=gb_r.at[1],
            device_id=(w1p,), device_id_type=pl.DeviceIdType.MESH,
        )
        g1t.start()
        g1b.start()
        g1t.wait_recv()
        g1b.wait_recv()

        for t in range(NQ - 1):
            ct_ = (q + 1 - t) % NQ
            cb_ = (q - 1 + t) % NQ
            rp = pltpu.make_async_remote_copy(
                src_ref=out_ref.at[pl.ds(ct_ * chunk, half), :],
                dst_ref=out_ref.at[pl.ds(ct_ * chunk, half), :],
                send_sem=ap_s.at[t], recv_sem=ap_r.at[t],
                device_id=(right,), device_id_type=pl.DeviceIdType.MESH,
            )
            rm = pltpu.make_async_remote_copy(
                src_ref=out_ref.at[pl.ds(cb_ * chunk + half, half), :],
                dst_ref=out_ref.at[pl.ds(cb_ * chunk + half, half), :],
                send_sem=am_s.at[t], recv_sem=am_r.at[t],
                device_id=(left,), device_id_type=pl.DeviceIdType.MESH,
            )
            rp.start()
            rm.start()
            rp.wait_recv()
            rm.wait_recv()
            rp.wait_send()
            rm.wait_send()

        for d in [e1t, e1b, e2t, e2b, g1t, g1b]:
            d.wait_send()

    nsteps = NQ - 1
    return pl.pallas_call(
        body,
        out_shape=jax.ShapeDtypeStruct((m, n), jnp.float32),
        in_specs=[
            pl.BlockSpec(memory_space=pltpu.VMEM),
            pl.BlockSpec(memory_space=pltpu.VMEM),
            pl.BlockSpec(memory_space=pltpu.VMEM),
        ],
        out_specs=pl.BlockSpec(memory_space=pltpu.VMEM),
        scratch_shapes=[
            pltpu.VMEM((NQ, half, n), jnp.float32),
            pltpu.VMEM((NQ, half, n), jnp.float32),
            pltpu.VMEM((nsteps, half, n), jnp.float32),
            pltpu.VMEM((nsteps, half, n), jnp.float32),
            pltpu.VMEM((2, half // 2, n), jnp.float32),
            pltpu.VMEM((2, half // 2, n), jnp.float32),
            pltpu.SemaphoreType.DMA((nsteps,)),
            pltpu.SemaphoreType.DMA((nsteps,)),
            pltpu.SemaphoreType.DMA((nsteps,)),
            pltpu.SemaphoreType.DMA((nsteps,)),
            pltpu.SemaphoreType.DMA((2,)),
            pltpu.SemaphoreType.DMA((2,)),
            pltpu.SemaphoreType.DMA((2,)),
            pltpu.SemaphoreType.DMA((2,)),
            pltpu.SemaphoreType.DMA((2,)),
            pltpu.SemaphoreType.DMA((2,)),
            pltpu.SemaphoreType.DMA((2,)),
            pltpu.SemaphoreType.DMA((2,)),
            pltpu.SemaphoreType.DMA((nsteps,)),
            pltpu.SemaphoreType.DMA((nsteps,)),
            pltpu.SemaphoreType.DMA((nsteps,)),
            pltpu.SemaphoreType.DMA((nsteps,)),
        ],
        compiler_params=pltpu.CompilerParams(collective_id=0),
    )(x, W1, W2)


# device time: 85930 ns/iter; 1.0487x vs baseline; 1.0328x over previous
import jax
import jax.numpy as jnp
from jax import lax
from jax.experimental import pallas as pl
from jax.experimental.pallas import tpu as pltpu

N_DEV = 16
NQ = 4


def kernel(x, W1, W2):
    m, k = x.shape
    _, h = W1.shape
    _, n = W2.shape
    chunk = m // NQ
    half = chunk // 2

    def body(x_ref, w1_ref, w2_ref, out_ref,
             ptop, pbot, ctop, cbot, zb1, zb2,
             sp_s, sp_r, sm_s, sm_r,
             za_s, za_r, zb_s, zb_r,
             ga_s, ga_r, gb_s, gb_r,
             qa_s, qa_r,
             ap_s, ap_r, am_s, am_r):
        me = lax.axis_index("i")
        z = me // NQ
        q = me % NQ
        right = z * NQ + (q + 1) % NQ
        left = z * NQ + (q - 1) % NQ
        w1p = me ^ 4
        w2p = me ^ 8

        barrier_sem = pltpu.get_barrier_semaphore()
        for nbr in [left, right, w1p, w2p]:
            pl.semaphore_signal(
                barrier_sem, inc=1,
                device_id=(nbr,), device_id_type=pl.DeviceIdType.MESH,
            )
        pl.semaphore_wait(barrier_sem, 4)

        def compute_top(c):
            xs = x_ref[pl.ds(c * chunk, half), :]
            hm = jnp.maximum(
                jnp.dot(xs, w1_ref[...], preferred_element_type=jnp.float32), 0.0)
            ptop[c] = jnp.dot(hm, w2_ref[...], preferred_element_type=jnp.float32)

        def compute_bot(c):
            xs = x_ref[pl.ds(c * chunk + half, half), :]
            hm = jnp.maximum(
                jnp.dot(xs, w1_ref[...], preferred_element_type=jnp.float32), 0.0)
            pbot[c] = jnp.dot(hm, w2_ref[...], preferred_element_type=jnp.float32)

        compute_top(q)

        for s in range(NQ - 1):
            st = (q - s) % NQ
            rt = (q - s - 1) % NQ
            sb = (q + s) % NQ
            rb = (q + s + 1) % NQ
            rp = pltpu.make_async_remote_copy(
                src_ref=ptop.at[st], dst_ref=ctop.at[s],
                send_sem=sp_s.at[s], recv_sem=sp_r.at[s],
                device_id=(right,), device_id_type=pl.DeviceIdType.MESH,
            )
            rm = pltpu.make_async_remote_copy(
                src_ref=pbot.at[sb], dst_ref=cbot.at[s],
                send_sem=sm_s.at[s], recv_sem=sm_r.at[s],
                device_id=(left,), device_id_type=pl.DeviceIdType.MESH,
            )
            rp.start()
            if s == 0:
                compute_bot(q)
            rm.start()
            if s == 0:
                compute_top((q - 1) % NQ)
                compute_bot((q + 1) % NQ)
            if s < NQ - 2:
                compute_top((q - s - 2) % NQ)
                compute_bot((q + s + 2) % NQ)
            rp.wait_recv()
            rm.wait_recv()
            ptop[rt] = ptop[rt] + ctop[s]
            pbot[rb] = pbot[rb] + cbot[s]
            rp.wait_send()
            rm.wait_send()

        ownt = (q + 1) % NQ
        ownb = (q - 1) % NQ

        koff = jnp.where(z % 2 == 0, 0, half // 2)
        soff = (half // 2) - koff
        e1t = pltpu.make_async_remote_copy(
            src_ref=ptop.at[ownt, pl.ds(soff, half // 2), :],
            dst_ref=zb1.at[0],
            send_sem=za_s.at[0], recv_sem=za_r.at[0],
            device_id=(w1p,), device_id_type=pl.DeviceIdType.MESH,
        )
        e1b = pltpu.make_async_remote_copy(
            src_ref=pbot.at[ownb, pl.ds(soff, half // 2), :],
            dst_ref=zb1.at[1],
            send_sem=za_s.at[1], recv_sem=za_r.at[1],
            device_id=(w1p,), device_id_type=pl.DeviceIdType.MESH,
        )
        e1t.start()
        e1b.start()
        e1t.wait_recv()
        e1b.wait_recv()
        ptop[ownt, pl.ds(koff, half // 2), :] = (
            ptop[ownt, pl.ds(koff, half // 2), :] + zb1[0])
        pbot[ownb, pl.ds(koff, half // 2), :] = (
            pbot[ownb, pl.ds(koff, half // 2), :] + zb1[1])

        e2t = pltpu.make_async_remote_copy(
            src_ref=ptop.at[ownt, pl.ds(koff, half // 2), :],
            dst_ref=zb2.at[0],
            send_sem=zb_s.at[0], recv_sem=zb_r.at[0],
            device_id=(w2p,), device_id_type=pl.DeviceIdType.MESH,
        )
        e2b = pltpu.make_async_remote_copy(
            src_ref=pbot.at[ownb, pl.ds(koff, half // 2), :],
            dst_ref=zb2.at[1],
            send_sem=zb_s.at[1], recv_sem=zb_r.at[1],
            device_id=(w2p,), device_id_type=pl.DeviceIdType.MESH,
        )
        e2t.start()
        e2b.start()
        e2t.wait_recv()
        e2b.wait_recv()
        ptop[ownt, pl.ds(koff, half // 2), :] = (
            ptop[ownt, pl.ds(koff, half // 2), :] + zb2[0])
        pbot[ownb, pl.ds(koff, half // 2), :] = (
            pbot[ownb, pl.ds(koff, half // 2), :] + zb2[1])

        trow = ownt * chunk + koff
        brow = ownb * chunk + half + koff
        out_ref[pl.ds(trow, half // 2), :] = ptop[ownt, pl.ds(koff, half // 2), :]
        out_ref[pl.ds(brow, half // 2), :] = pbot[ownb, pl.ds(koff, half // 2), :]

        soff_t = ownt * chunk + koff
        soff_b = ownb * chunk + half + koff
        aA_t = pltpu.make_async_remote_copy(
            src_ref=out_ref.at[pl.ds(soff_t, half // 2), :],
            dst_ref=out_ref.at[pl.ds(soff_t, half // 2), :],
            send_sem=qa_s.at[0], recv_sem=qa_r.at[0],
            device_id=(right,), device_id_type=pl.DeviceIdType.MESH,
        )
        aA_b = pltpu.make_async_remote_copy(
            src_ref=out_ref.at[pl.ds(soff_b, half // 2), :],
            dst_ref=out_ref.at[pl.ds(soff_b, half // 2), :],
            send_sem=qa_s.at[1], recv_sem=qa_r.at[1],
            device_id=(left,), device_id_type=pl.DeviceIdType.MESH,
        )
        aA_t.start()
        aA_b.start()

        t64 = ownt * chunk + koff
        b64 = ownb * chunk + half + koff
        g1t = pltpu.make_async_remote_copy(
            src_ref=out_ref.at[pl.ds(t64, half // 2), :],
            dst_ref=out_ref.at[pl.ds(t64, half // 2), :],
            send_sem=gb_s.at[0], recv_sem=gb_r.at[0],
            device_id=(w1p,), device_id_type=pl.DeviceIdType.MESH,
        )
        g1b = pltpu.make_async_remote_copy(
            src_ref=out_ref.at[pl.ds(b64, half // 2), :],
            dst_ref=out_ref.at[pl.ds(b64, half // 2), :],
            send_sem=gb_s.at[1], recv_sem=gb_r.at[1],
            device_id=(w1p,), device_id_type=pl.DeviceIdType.MESH,
        )
        g1t.start()
        g1b.start()
        g1t.wait_recv()
        g1b.wait_recv()

        for t in range(NQ - 1):
            ct_ = (q + 1 - t) % NQ
            cb_ = (q - 1 + t) % NQ
            if t == 0:
                top_rows = pl.ds(ownt * chunk + soff, half // 2)
                bot_rows = pl.ds(ownb * chunk + half + soff, half // 2)
            else:
                top_rows = pl.ds(ct_ * chunk, half)
                bot_rows = pl.ds(cb_ * chunk + half, half)
            rp = pltpu.make_async_remote_copy(
                src_ref=out_ref.at[top_rows, :],
                dst_ref=out_ref.at[top_rows, :],
                send_sem=ap_s.at[t], recv_sem=ap_r.at[t],
                device_id=(right,), device_id_type=pl.DeviceIdType.MESH,
            )
            rm = pltpu.make_async_remote_copy(
                src_ref=out_ref.at[bot_rows, :],
                dst_ref=out_ref.at[bot_rows, :],
                send_sem=am_s.at[t], recv_sem=am_r.at[t],
                device_id=(left,), device_id_type=pl.DeviceIdType.MESH,
            )
            rp.start()
            rm.start()
            rp.wait_recv()
            rm.wait_recv()
            if t == 0:
                aA_t.wait_recv()
                aA_b.wait_recv()
            rp.wait_send()
            rm.wait_send()

        aA_t.wait_send()
        aA_b.wait_send()

        for d in [e1t, e1b, e2t, e2b, g1t, g1b]:
            d.wait_send()

    nsteps = NQ - 1
    return pl.pallas_call(
        body,
        out_shape=jax.ShapeDtypeStruct((m, n), jnp.float32),
        in_specs=[
            pl.BlockSpec(memory_space=pltpu.VMEM),
            pl.BlockSpec(memory_space=pltpu.VMEM),
            pl.BlockSpec(memory_space=pltpu.VMEM),
        ],
        out_specs=pl.BlockSpec(memory_space=pltpu.VMEM),
        scratch_shapes=[
            pltpu.VMEM((NQ, half, n), jnp.float32),
            pltpu.VMEM((NQ, half, n), jnp.float32),
            pltpu.VMEM((nsteps, half, n), jnp.float32),
            pltpu.VMEM((nsteps, half, n), jnp.float32),
            pltpu.VMEM((2, half // 2, n), jnp.float32),
            pltpu.VMEM((2, half // 2, n), jnp.float32),
            pltpu.SemaphoreType.DMA((nsteps,)),
            pltpu.SemaphoreType.DMA((nsteps,)),
            pltpu.SemaphoreType.DMA((nsteps,)),
            pltpu.SemaphoreType.DMA((nsteps,)),
            pltpu.SemaphoreType.DMA((2,)),
            pltpu.SemaphoreType.DMA((2,)),
            pltpu.SemaphoreType.DMA((2,)),
            pltpu.SemaphoreType.DMA((2,)),
            pltpu.SemaphoreType.DMA((2,)),
            pltpu.SemaphoreType.DMA((2,)),
            pltpu.SemaphoreType.DMA((2,)),
            pltpu.SemaphoreType.DMA((2,)),
            pltpu.SemaphoreType.DMA((2,)),
            pltpu.SemaphoreType.DMA((2,)),
            pltpu.SemaphoreType.DMA((nsteps,)),
            pltpu.SemaphoreType.DMA((nsteps,)),
            pltpu.SemaphoreType.DMA((nsteps,)),
            pltpu.SemaphoreType.DMA((nsteps,)),
        ],
        compiler_params=pltpu.CompilerParams(collective_id=0),
    )(x, W1, W2)
